# baseline (speedup 1.0000x reference)
import functools

import jax
import jax.numpy as jnp
from jax import lax
from jax.experimental import pallas as pl
from jax.experimental.pallas import tpu as pltpu
from jax.experimental.pallas import tpu_sc as plsc

_LANES = 16
_CHUNK = 128


def _sc_embed(table, idxp, g1f, g2f, *, batch, n_tokens, seq_tail, dim):
    info = plsc.get_sparse_core_info()
    nw = info.num_cores * info.num_subcores
    nb = batch // nw
    n_chunks = idxp.shape[1]
    n_pref = n_tokens * dim
    buf_rows = n_tokens + n_chunks * _CHUNK
    out_rows = n_tokens + seq_tail
    mesh = plsc.VectorSubcoreMesh(core_axis_name="c", subcore_axis_name="s")

    @functools.partial(
        pl.kernel,
        mesh=mesh,
        compiler_params=pltpu.CompilerParams(use_tc_tiling_on_sc=False),
        out_type=jax.ShapeDtypeStruct((batch, out_rows, dim), jnp.float32),
        scratch_types=[
            pltpu.VMEM((buf_rows, dim), jnp.float32),
            pltpu.VMEM((n_chunks, _CHUNK), jnp.int32),
            pltpu.VMEM((n_pref,), jnp.float32),
            pltpu.VMEM((n_pref,), jnp.float32),
            pltpu.SemaphoreType.DMA,
        ],
    )
    def k(table_hbm, idx_hbm, g1_hbm, g2_hbm, out_hbm, buf, idxv, g1v, g2v, sem):
        wid = lax.axis_index("s") * info.num_cores + lax.axis_index("c")

        def body(i, carry):
            b = wid * nb + i
            pltpu.sync_copy(idx_hbm.at[b], idxv)
            pltpu.sync_copy(g1_hbm.at[b], g1v)
            pltpu.sync_copy(g2_hbm.at[b], g2v)
            cps = [
                pltpu.async_copy(
                    table_hbm.at[idxv.at[j]],
                    buf.at[pl.ds(n_tokens + j * _CHUNK, _CHUNK)],
                    sem,
                )
                for j in range(n_chunks)
            ]
            per_row = dim // _LANES
            for j in range(n_pref // _LANES):
                a = g1v[pl.ds(j * _LANES, _LANES)]
                c = g2v[pl.ds(j * _LANES, _LANES)]
                buf[j // per_row, pl.ds((j % per_row) * _LANES, _LANES)] = a / (a + c)
            for cp in cps:
                cp.wait()
            pltpu.sync_copy(buf.at[pl.ds(0, out_rows)], out_hbm.at[b])
            return carry

        lax.fori_loop(0, nb, body, 0)

    return k(table, idxp, g1f, g2f)


def kernel(tokens, table, alpha, beta):
    n_tokens = alpha.shape[0]
    batch, seq = tokens.shape
    dim = table.shape[1]
    seq_tail = seq - n_tokens

    key = jax.random.key(42)
    ka, kb = jax.random.split(key)
    g1 = jax.random.gamma(ka, alpha, shape=(batch,) + alpha.shape)
    g2 = jax.random.gamma(kb, beta, shape=(batch,) + beta.shape)

    tail = tokens[:, n_tokens:]
    pad = (-seq_tail) % _CHUNK
    idxp = jnp.pad(tail, ((0, 0), (0, pad))).reshape(batch, -1, _CHUNK)
    return _sc_embed(
        table,
        idxp,
        g1.reshape(batch, n_tokens * dim),
        g2.reshape(batch, n_tokens * dim),
        batch=batch,
        n_tokens=n_tokens,
        seq_tail=seq_tail,
        dim=dim,
    )


# speedup vs baseline: 1.4578x; 1.4578x over previous
import functools

import jax
import jax.numpy as jnp
from jax import lax
from jax.experimental import pallas as pl
from jax.experimental.pallas import tpu as pltpu
from jax.experimental.pallas import tpu_sc as plsc

_LANES = 16
_CHUNK = 128
_G = 2
_NBUF = 3


def _sc_embed(table, idxp, g1f, g2f, *, batch, n_tokens, seq_tail, dim):
    info = plsc.get_sparse_core_info()
    nw = info.num_cores * info.num_subcores
    nb = batch // nw
    n_stage = nb // _G
    n_pref = n_tokens * dim
    out_rows = n_tokens + seq_tail
    tail_a = _CHUNK
    tail_b = -(-(seq_tail - _CHUNK) // 8) * 8
    spill = tail_b - (seq_tail - _CHUNK)
    assert spill <= n_tokens, "spill rows must stay within the next prefix"
    per_row = dim // _LANES
    mesh = plsc.VectorSubcoreMesh(core_axis_name="c", subcore_axis_name="s")

    @functools.partial(
        pl.kernel,
        mesh=mesh,
        compiler_params=pltpu.CompilerParams(use_tc_tiling_on_sc=False),
        out_type=jax.ShapeDtypeStruct((batch * out_rows, dim), jnp.float32),
        scratch_types=[
            pltpu.VMEM((_NBUF, _G * out_rows + spill, dim), jnp.float32),
            pltpu.VMEM((nb, 2, _CHUNK), jnp.int32),
            pltpu.VMEM((nb, n_pref), jnp.float32),
            pltpu.VMEM((nb, n_pref), jnp.float32),
            pltpu.SemaphoreType.DMA((_NBUF,)),
            pltpu.SemaphoreType.DMA((_NBUF,)),
        ],
    )
    def k(table_hbm, idx_hbm, g1_hbm, g2_hbm, out_hbm, bufs, idxv, g1v, g2v, gsem, osem):
        wid = lax.axis_index("s") * info.num_cores + lax.axis_index("c")
        b0 = wid * nb

        pltpu.sync_copy(idx_hbm.at[pl.ds(b0, nb)], idxv)
        pltpu.sync_copy(g1_hbm.at[pl.ds(b0, nb)], g1v)
        pltpu.sync_copy(g2_hbm.at[pl.ds(b0, nb)], g2v)

        def issue_stage(s):
            kb = s % _NBUF
            for g in range(_G):
                row = g * out_rows
                pltpu.async_copy(
                    table_hbm.at[idxv.at[_G * s + g, 0]],
                    bufs.at[kb, pl.ds(row + n_tokens, tail_a)],
                    gsem.at[kb],
                )
                pltpu.async_copy(
                    table_hbm.at[idxv.at[_G * s + g, 1, pl.ds(0, tail_b)]],
                    bufs.at[kb, pl.ds(row + n_tokens + tail_a, tail_b)],
                    gsem.at[kb],
                )

        def wait_gathers(kb):
            for g in range(_G):
                row = g * out_rows
                pltpu.make_async_copy(
                    table_hbm.at[pl.ds(0, tail_a)],
                    bufs.at[kb, pl.ds(row + n_tokens, tail_a)],
                    gsem.at[kb],
                ).wait()
                pltpu.make_async_copy(
                    table_hbm.at[pl.ds(0, tail_b)],
                    bufs.at[kb, pl.ds(row + n_tokens + tail_a, tail_b)],
                    gsem.at[kb],
                ).wait()

        def wait_out(kb):
            pltpu.make_async_copy(
                bufs.at[kb, pl.ds(0, _G * out_rows)],
                out_hbm.at[pl.ds(0, _G * out_rows)],
                osem.at[kb],
            ).wait()

        issue_stage(0)

        def body(s, carry):
            kb = s % _NBUF
            wait_gathers(kb)
            for g in range(_G):
                for j in range(n_pref // _LANES):
                    a = g1v[_G * s + g, pl.ds(j * _LANES, _LANES)]
                    c = g2v[_G * s + g, pl.ds(j * _LANES, _LANES)]
                    bufs[kb, g * out_rows + j // per_row, pl.ds((j % per_row) * _LANES, _LANES)] = a / (a + c)
            pltpu.async_copy(
                bufs.at[kb, pl.ds(0, _G * out_rows)],
                out_hbm.at[pl.ds((b0 + _G * s) * out_rows, _G * out_rows)],
                osem.at[kb],
            )

            @pl.when(s + 1 < n_stage)
            def _():
                @pl.when(s >= _NBUF - 1)
                def _():
                    wait_out((s + 1) % _NBUF)

                issue_stage(s + 1)

            return carry

        lax.fori_loop(0, n_stage, body, 0)
        for t in range(_NBUF - 1):
            wait_out((n_stage - 1 - t) % _NBUF)

    return k(table, idxp, g1f, g2f)


def kernel(tokens, table, alpha, beta):
    n_tokens = alpha.shape[0]
    batch, seq = tokens.shape
    dim = table.shape[1]
    seq_tail = seq - n_tokens

    key = jax.random.key(42)
    ka, kb = jax.random.split(key)
    g1 = jax.random.gamma(ka, alpha, shape=(batch,) + alpha.shape)
    g2 = jax.random.gamma(kb, beta, shape=(batch,) + beta.shape)

    tail = tokens[:, n_tokens:]
    pad = (-seq_tail) % _CHUNK
    idxp = jnp.pad(tail, ((0, 0), (0, pad))).reshape(batch, -1, _CHUNK)
    flat = _sc_embed(
        table,
        idxp,
        g1.reshape(batch, n_tokens * dim),
        g2.reshape(batch, n_tokens * dim),
        batch=batch,
        n_tokens=n_tokens,
        seq_tail=seq_tail,
        dim=dim,
    )
    return flat.reshape(batch, n_tokens + seq_tail, dim)
